# baseline (speedup 1.0000x reference)
import functools

import jax
import jax.numpy as jnp
from jax import lax
from jax.experimental import pallas as pl
from jax.experimental.pallas import tpu as pltpu
from jax.experimental.pallas import tpu_sc as plsc

_N = 10000
_E = 320000
_D = 128

_NC = 2
_NS = 16
_L = 16

_EPT = _E // (_NC * _NS)
_K = 80
_NCHUNK = _EPT // _K
_NPAIR = _NCHUNK // 2
_NPAD = 10240
_RPT = _NPAD // _NS
_STG = 80
_NSTG = _RPT // _STG


def _sc_segment_sum(ego, src, dst, ev, zeros):
    mesh = plsc.VectorSubcoreMesh(core_axis_name="c", subcore_axis_name="s")

    @functools.partial(
        pl.kernel,
        mesh=mesh,
        out_type=jax.ShapeDtypeStruct((_NC, _NPAD, _D), jnp.float32),
        scratch_types=[
            pltpu.VMEM((_EPT,), jnp.int32),
            pltpu.VMEM((_EPT + _L,), jnp.float32),
            pltpu.VMEM((_K,), jnp.int32),
            pltpu.VMEM((_K,), jnp.int32),
            pltpu.VMEM((_K, _D), jnp.float32),
            pltpu.VMEM((_K, _D), jnp.float32),
            pltpu.VMEM_SHARED((_NPAD, _D), jnp.float32),
            pltpu.SemaphoreType.DMA,
            pltpu.SemaphoreType.DMA,
            pltpu.SemaphoreType.DMA,
            pltpu.SemaphoreType.DMA,
            pltpu.SemaphoreType.DMA,
            pltpu.SemaphoreType.DMA,
        ],
    )
    def body(ego_hbm, src_hbm, dst_hbm, ev_hbm, z_hbm, out_hbm,
             src_v, ev_v, dst_a, dst_b, rows_a, rows_b, acc,
             sg_a, sg_b, ss_a, ss_b, sd_a, sd_b):
        c = lax.axis_index("c")
        s = lax.axis_index("s")
        tile = c * _NS + s
        ebase = tile * _EPT

        pltpu.sync_copy(z_hbm, acc.at[pl.ds(s * _RPT, _RPT)])

        pltpu.sync_copy(src_hbm.at[pl.ds(ebase, _EPT)], src_v)
        pltpu.sync_copy(ev_hbm.at[pl.ds(ebase, _EPT)], ev_v.at[pl.ds(0, _EPT)])

        plsc.subcore_barrier()

        def scale(rows, off):
            @plsc.parallel_loop(0, _K, unroll=8)
            def edge(e):
                ve = ev_v[pl.ds(off + e, _L)]
                bc = jnp.full((_L,), ve[0], jnp.float32)
                for j in range(_D // _L):
                    sl = pl.ds(j * _L, _L)
                    rows[e, sl] = rows[e, sl] * bc

        kh = _K // 2

        def gather_start(off, rows, sem):
            pltpu.async_copy(ego_hbm.at[src_v.at[pl.ds(off, kh)]],
                             rows.at[pl.ds(0, kh)], sem)
            pltpu.async_copy(ego_hbm.at[src_v.at[pl.ds(off + kh, kh)]],
                             rows.at[pl.ds(kh, kh)], sem)

        def gather_wait(off, rows, sem):
            pltpu.make_async_copy(
                ego_hbm.at[src_v.at[pl.ds(off, kh)]],
                rows.at[pl.ds(0, kh)], sem).wait()
            pltpu.make_async_copy(
                ego_hbm.at[src_v.at[pl.ds(off + kh, kh)]],
                rows.at[pl.ds(kh, kh)], sem).wait()

        def dst_start(off, buf, sem):
            pltpu.async_copy(dst_hbm.at[pl.ds(ebase + off, _K)], buf, sem)

        def dst_wait(off, buf, sem):
            pltpu.make_async_copy(
                dst_hbm.at[pl.ds(ebase + off, _K)], buf, sem).wait()

        def half(off_cur, off_nxt, rows_cur, rows_oth, dst_cur, dst_oth,
                 sg_cur, sg_oth, ss_cur, ss_oth, sd_cur, sd_oth, first):
            gather_wait(off_cur, rows_cur, sg_cur)
            if not first:
                pltpu.make_async_copy(rows_oth, acc.at[dst_oth], ss_oth).wait()
            dst_start(off_nxt, dst_oth, sd_oth)
            gather_start(off_nxt, rows_oth, sg_oth)
            scale(rows_cur, off_cur)
            dst_wait(off_cur, dst_cur, sd_cur)
            pltpu.async_copy(rows_cur, acc.at[dst_cur], ss_cur, add=True)

        def do_pair(j, first):
            off0 = j * (2 * _K)
            half(off0, off0 + _K, rows_a, rows_b, dst_a, dst_b,
                 sg_a, sg_b, ss_a, ss_b, sd_a, sd_b, first)
            half(off0 + _K, off0 + 2 * _K, rows_b, rows_a, dst_b, dst_a,
                 sg_b, sg_a, ss_b, ss_a, sd_b, sd_a, False)

        gather_start(0, rows_a, sg_a)
        dst_start(0, dst_a, sd_a)
        do_pair(0, True)

        def pair_loop(j, carry):
            do_pair(j, False)
            return carry
        lax.fori_loop(1, _NPAIR, pair_loop, 0)

        offl = (_NCHUNK - 1) * _K
        gather_wait(offl, rows_a, sg_a)
        pltpu.make_async_copy(rows_b, acc.at[dst_b], ss_b).wait()
        scale(rows_a, offl)
        dst_wait(offl, dst_a, sd_a)
        pltpu.sync_copy(rows_a, acc.at[dst_a], add=True)

        plsc.subcore_barrier()

        for t in range(_NSTG):
            r0 = s * _RPT + t * _STG
            pltpu.sync_copy(acc.at[pl.ds(r0, _STG)], rows_a)
            pltpu.sync_copy(rows_a, out_hbm.at[c, pl.ds(r0, _STG)])

    return body(ego, src, dst, ev, zeros)


def _tc_combine(ego, partial, W1, b1, W2, b2):
    br = 1000
    grid = (_N // br,)

    def body(ego_ref, p_ref, w1_ref, b1_ref, w2_ref, b2_ref, out_ref):
        side = p_ref[0] + p_ref[1]
        e = ego_ref[...]
        x1 = jnp.dot(e + side, w1_ref[...],
                     preferred_element_type=jnp.float32) + b1_ref[...]
        x2 = jnp.dot(e * side, w2_ref[...],
                     preferred_element_type=jnp.float32) + b2_ref[...]
        out_ref[...] = (jnp.where(x1 >= 0, x1, 0.01 * x1)
                        + jnp.where(x2 >= 0, x2, 0.01 * x2))

    return pl.pallas_call(
        body,
        grid=grid,
        in_specs=[
            pl.BlockSpec((br, _D), lambda i: (i, 0)),
            pl.BlockSpec((_NC, br, _D), lambda i: (0, i, 0)),
            pl.BlockSpec((_D, _D), lambda i: (0, 0)),
            pl.BlockSpec((1, _D), lambda i: (0, 0)),
            pl.BlockSpec((_D, _D), lambda i: (0, 0)),
            pl.BlockSpec((1, _D), lambda i: (0, 0)),
        ],
        out_specs=pl.BlockSpec((br, _D), lambda i: (i, 0)),
        out_shape=jax.ShapeDtypeStruct((_N, _D), jnp.float32),
    )(ego, partial, W1, b1.reshape(1, _D), W2, b2.reshape(1, _D))


def kernel(ego_embeddings, edge_index, edge_values, W1, b1, W2, b2):
    src = edge_index[0].astype(jnp.int32)
    dst = edge_index[1].astype(jnp.int32)
    ev = edge_values.astype(jnp.float32)
    zeros = jnp.zeros((_RPT, _D), jnp.float32)
    partial = _sc_segment_sum(ego_embeddings, src, dst, ev, zeros)
    return _tc_combine(ego_embeddings, partial, W1, b1, W2, b2)


# speedup vs baseline: 1.0142x; 1.0142x over previous
import functools

import jax
import jax.numpy as jnp
from jax import lax
from jax.experimental import pallas as pl
from jax.experimental.pallas import tpu as pltpu
from jax.experimental.pallas import tpu_sc as plsc

_N = 10000
_E = 320000
_D = 128

_NC = 2
_NS = 16
_L = 16

_EPT = _E // (_NC * _NS)
_K = 80
_NCHUNK = _EPT // _K
_NPAIR = _NCHUNK // 2
_NPAD = 10240
_RPT = _NPAD // _NS
_STG = 80
_NSTG = _RPT // _STG


def _sc_segment_sum(ego, src, dst, ev, zeros):
    mesh = plsc.VectorSubcoreMesh(core_axis_name="c", subcore_axis_name="s")

    @functools.partial(
        pl.kernel,
        mesh=mesh,
        out_type=jax.ShapeDtypeStruct((_NC, _NPAD, _D), jnp.float32),
        scratch_types=[
            pltpu.VMEM((_EPT,), jnp.int32),
            pltpu.VMEM((_EPT + _L,), jnp.float32),
            pltpu.VMEM((_K,), jnp.int32),
            pltpu.VMEM((_K,), jnp.int32),
            pltpu.VMEM((_K, _D), jnp.float32),
            pltpu.VMEM((_K, _D), jnp.float32),
            pltpu.VMEM_SHARED((_NPAD, _D), jnp.float32),
            pltpu.SemaphoreType.DMA,
            pltpu.SemaphoreType.DMA,
            pltpu.SemaphoreType.DMA,
            pltpu.SemaphoreType.DMA,
            pltpu.SemaphoreType.DMA,
            pltpu.SemaphoreType.DMA,
        ],
    )
    def body(ego_hbm, src_hbm, dst_hbm, ev_hbm, z_hbm, out_hbm,
             src_v, ev_v, dst_a, dst_b, rows_a, rows_b, acc,
             sg_a, sg_b, ss_a, ss_b, sd_a, sd_b):
        c = lax.axis_index("c")
        s = lax.axis_index("s")
        tile = c * _NS + s
        ebase = tile * _EPT

        pltpu.sync_copy(z_hbm, acc.at[pl.ds(s * _RPT, _RPT)])

        pltpu.sync_copy(src_hbm.at[pl.ds(ebase, _EPT)], src_v)
        pltpu.sync_copy(ev_hbm.at[pl.ds(ebase, _EPT)], ev_v.at[pl.ds(0, _EPT)])

        plsc.subcore_barrier()

        def scale(rows, off):
            @plsc.parallel_loop(0, _K, unroll=8)
            def edge(e):
                ve = ev_v[pl.ds(off + e, _L)]
                bc = jnp.full((_L,), ve[0], jnp.float32)
                for j in range(_D // _L):
                    sl = pl.ds(j * _L, _L)
                    rows[e, sl] = rows[e, sl] * bc

        def gather_start(off, rows, sem):
            pltpu.async_copy(ego_hbm.at[src_v.at[pl.ds(off, _K)]], rows, sem)

        def gather_wait(off, rows, sem):
            pltpu.make_async_copy(
                ego_hbm.at[src_v.at[pl.ds(off, _K)]], rows, sem).wait()

        def dst_start(off, buf, sem):
            pltpu.async_copy(dst_hbm.at[pl.ds(ebase + off, _K)], buf, sem)

        def dst_wait(off, buf, sem):
            pltpu.make_async_copy(
                dst_hbm.at[pl.ds(ebase + off, _K)], buf, sem).wait()

        def half(off_cur, off_nxt, rows_cur, rows_oth, dst_cur, dst_oth,
                 sg_cur, sg_oth, ss_cur, ss_oth, sd_cur, sd_oth, first):
            gather_wait(off_cur, rows_cur, sg_cur)
            if not first:
                pltpu.make_async_copy(rows_oth, acc.at[dst_oth], ss_oth).wait()
            dst_start(off_nxt, dst_oth, sd_oth)
            gather_start(off_nxt, rows_oth, sg_oth)
            scale(rows_cur, off_cur)
            dst_wait(off_cur, dst_cur, sd_cur)
            pltpu.async_copy(rows_cur, acc.at[dst_cur], ss_cur, add=True)

        def do_pair(j, first):
            off0 = j * (2 * _K)
            half(off0, off0 + _K, rows_a, rows_b, dst_a, dst_b,
                 sg_a, sg_b, ss_a, ss_b, sd_a, sd_b, first)
            half(off0 + _K, off0 + 2 * _K, rows_b, rows_a, dst_b, dst_a,
                 sg_b, sg_a, ss_b, ss_a, sd_b, sd_a, False)

        gather_start(0, rows_a, sg_a)
        dst_start(0, dst_a, sd_a)
        do_pair(0, True)

        def pair_loop(j, carry):
            do_pair(j, False)
            return carry
        lax.fori_loop(1, _NPAIR, pair_loop, 0)

        offl = (_NCHUNK - 1) * _K
        gather_wait(offl, rows_a, sg_a)
        pltpu.make_async_copy(rows_b, acc.at[dst_b], ss_b).wait()
        scale(rows_a, offl)
        dst_wait(offl, dst_a, sd_a)
        pltpu.sync_copy(rows_a, acc.at[dst_a], add=True)

        plsc.subcore_barrier()

        for t in range(_NSTG):
            r0 = s * _RPT + t * _STG
            pltpu.sync_copy(acc.at[pl.ds(r0, _STG)], rows_a)
            pltpu.sync_copy(rows_a, out_hbm.at[c, pl.ds(r0, _STG)])

    return body(ego, src, dst, ev, zeros)


def _tc_combine(ego, partial, W1, b1, W2, b2):
    br = 2000
    grid = (_N // br,)

    def body(ego_ref, p_ref, w1_ref, b1_ref, w2_ref, b2_ref, out_ref):
        side = p_ref[0] + p_ref[1]
        e = ego_ref[...]
        x1 = jnp.dot(e + side, w1_ref[...],
                     preferred_element_type=jnp.float32) + b1_ref[...]
        x2 = jnp.dot(e * side, w2_ref[...],
                     preferred_element_type=jnp.float32) + b2_ref[...]
        out_ref[...] = (jnp.where(x1 >= 0, x1, 0.01 * x1)
                        + jnp.where(x2 >= 0, x2, 0.01 * x2))

    return pl.pallas_call(
        body,
        grid=grid,
        in_specs=[
            pl.BlockSpec((br, _D), lambda i: (i, 0)),
            pl.BlockSpec((_NC, br, _D), lambda i: (0, i, 0)),
            pl.BlockSpec((_D, _D), lambda i: (0, 0)),
            pl.BlockSpec((1, _D), lambda i: (0, 0)),
            pl.BlockSpec((_D, _D), lambda i: (0, 0)),
            pl.BlockSpec((1, _D), lambda i: (0, 0)),
        ],
        out_specs=pl.BlockSpec((br, _D), lambda i: (i, 0)),
        out_shape=jax.ShapeDtypeStruct((_N, _D), jnp.float32),
    )(ego, partial, W1, b1.reshape(1, _D), W2, b2.reshape(1, _D))


def kernel(ego_embeddings, edge_index, edge_values, W1, b1, W2, b2):
    src = edge_index[0].astype(jnp.int32)
    dst = edge_index[1].astype(jnp.int32)
    ev = edge_values.astype(jnp.float32)
    zeros = jnp.zeros((_RPT, _D), jnp.float32)
    partial = _sc_segment_sum(ego_embeddings, src, dst, ev, zeros)
    return _tc_combine(ego_embeddings, partial, W1, b1, W2, b2)


# speedup vs baseline: 1.0262x; 1.0118x over previous
import functools

import jax
import jax.numpy as jnp
from jax import lax
from jax.experimental import pallas as pl
from jax.experimental.pallas import tpu as pltpu
from jax.experimental.pallas import tpu_sc as plsc

_N = 10000
_E = 320000
_D = 128

_NC = 2
_NS = 16
_L = 16

_EPT = _E // (_NC * _NS)
_K = 80
_NCHUNK = _EPT // _K
_NPAIR = _NCHUNK // 2
_NPAD = 10240
_RPT = _NPAD // _NS
_STG = 80
_NSTG = _RPT // _STG


def _sc_segment_sum(ego, src, dst, ev, zeros):
    mesh = plsc.VectorSubcoreMesh(core_axis_name="c", subcore_axis_name="s")

    @functools.partial(
        pl.kernel,
        mesh=mesh,
        out_type=jax.ShapeDtypeStruct((_NC, _NPAD, _D), jnp.float32),
        scratch_types=[
            pltpu.VMEM((_EPT,), jnp.int32),
            pltpu.VMEM((_EPT + _L,), jnp.float32),
            pltpu.VMEM((_K,), jnp.int32),
            pltpu.VMEM((_K,), jnp.int32),
            pltpu.VMEM((_K, _D), jnp.float32),
            pltpu.VMEM((_K, _D), jnp.float32),
            pltpu.VMEM_SHARED((_NPAD, _D), jnp.float32),
            pltpu.SemaphoreType.DMA,
            pltpu.SemaphoreType.DMA,
            pltpu.SemaphoreType.DMA,
            pltpu.SemaphoreType.DMA,
            pltpu.SemaphoreType.DMA,
            pltpu.SemaphoreType.DMA,
        ],
    )
    def body(ego_hbm, src_hbm, dst_hbm, ev_hbm, z_hbm, out_hbm,
             src_v, ev_v, dst_a, dst_b, rows_a, rows_b, acc,
             sg_a, sg_b, ss_a, ss_b, sd_a, sd_b):
        c = lax.axis_index("c")
        s = lax.axis_index("s")
        tile = c * _NS + s
        ebase = tile * _EPT

        pltpu.sync_copy(z_hbm, acc.at[pl.ds(s * _RPT, _RPT)])

        pltpu.sync_copy(src_hbm.at[pl.ds(ebase, _EPT)], src_v)
        pltpu.sync_copy(ev_hbm.at[pl.ds(ebase, _EPT)], ev_v.at[pl.ds(0, _EPT)])

        plsc.subcore_barrier()

        def scale(rows, off):
            @plsc.parallel_loop(0, _K, unroll=8)
            def edge(e):
                ve = ev_v[pl.ds(off + e, _L)]
                bc = jnp.full((_L,), ve[0], jnp.float32)
                for j in range(_D // _L):
                    sl = pl.ds(j * _L, _L)
                    rows[e, sl] = rows[e, sl] * bc

        def gather_start(off, rows, sem):
            pltpu.async_copy(ego_hbm.at[src_v.at[pl.ds(off, _K)]], rows, sem)

        def gather_wait(off, rows, sem):
            pltpu.make_async_copy(
                ego_hbm.at[src_v.at[pl.ds(off, _K)]], rows, sem).wait()

        def dst_start(off, buf, sem):
            pltpu.async_copy(dst_hbm.at[pl.ds(ebase + off, _K)], buf, sem)

        def dst_wait(off, buf, sem):
            pltpu.make_async_copy(
                dst_hbm.at[pl.ds(ebase + off, _K)], buf, sem).wait()

        def half(off_cur, off_nxt, rows_cur, rows_oth, dst_cur, dst_oth,
                 sg_cur, sg_oth, ss_cur, ss_oth, sd_cur, sd_oth, first):
            gather_wait(off_cur, rows_cur, sg_cur)
            if not first:
                pltpu.make_async_copy(rows_oth, acc.at[dst_oth], ss_oth).wait()
            dst_start(off_nxt, dst_oth, sd_oth)
            gather_start(off_nxt, rows_oth, sg_oth)
            scale(rows_cur, off_cur)
            dst_wait(off_cur, dst_cur, sd_cur)
            pltpu.async_copy(rows_cur, acc.at[dst_cur], ss_cur, add=True)

        def do_pair(j, first):
            off0 = j * (2 * _K)
            half(off0, off0 + _K, rows_a, rows_b, dst_a, dst_b,
                 sg_a, sg_b, ss_a, ss_b, sd_a, sd_b, first)
            half(off0 + _K, off0 + 2 * _K, rows_b, rows_a, dst_b, dst_a,
                 sg_b, sg_a, ss_b, ss_a, sd_b, sd_a, False)

        gather_start(0, rows_a, sg_a)
        dst_start(0, dst_a, sd_a)
        do_pair(0, True)

        def pair_loop(j, carry):
            do_pair(j, False)
            return carry
        lax.fori_loop(1, _NPAIR, pair_loop, 0)

        offl = (_NCHUNK - 1) * _K
        gather_wait(offl, rows_a, sg_a)
        pltpu.make_async_copy(rows_b, acc.at[dst_b], ss_b).wait()
        scale(rows_a, offl)
        dst_wait(offl, dst_a, sd_a)
        pltpu.sync_copy(rows_a, acc.at[dst_a], add=True)

        plsc.subcore_barrier()

        rbase = s * _RPT
        pltpu.sync_copy(acc.at[pl.ds(rbase, _STG)], rows_a)
        for t in range(_NSTG):
            cur, nxt = (rows_a, rows_b) if t % 2 == 0 else (rows_b, rows_a)
            sw = ss_a if t % 2 == 0 else ss_b
            r0 = rbase + t * _STG
            pltpu.async_copy(cur, out_hbm.at[c, pl.ds(r0, _STG)], sw)
            if t + 1 < _NSTG:
                pltpu.sync_copy(acc.at[pl.ds(r0 + _STG, _STG)], nxt)
            pltpu.make_async_copy(cur, out_hbm.at[c, pl.ds(r0, _STG)], sw).wait()

    return body(ego, src, dst, ev, zeros)


def _tc_combine(ego, partial, W1, b1, W2, b2):
    br = 2000
    grid = (_N // br,)

    def body(ego_ref, p_ref, w1_ref, b1_ref, w2_ref, b2_ref, out_ref):
        side = p_ref[0] + p_ref[1]
        e = ego_ref[...]
        x1 = jnp.dot(e + side, w1_ref[...],
                     preferred_element_type=jnp.float32) + b1_ref[...]
        x2 = jnp.dot(e * side, w2_ref[...],
                     preferred_element_type=jnp.float32) + b2_ref[...]
        out_ref[...] = (jnp.where(x1 >= 0, x1, 0.01 * x1)
                        + jnp.where(x2 >= 0, x2, 0.01 * x2))

    return pl.pallas_call(
        body,
        grid=grid,
        in_specs=[
            pl.BlockSpec((br, _D), lambda i: (i, 0)),
            pl.BlockSpec((_NC, br, _D), lambda i: (0, i, 0)),
            pl.BlockSpec((_D, _D), lambda i: (0, 0)),
            pl.BlockSpec((1, _D), lambda i: (0, 0)),
            pl.BlockSpec((_D, _D), lambda i: (0, 0)),
            pl.BlockSpec((1, _D), lambda i: (0, 0)),
        ],
        out_specs=pl.BlockSpec((br, _D), lambda i: (i, 0)),
        out_shape=jax.ShapeDtypeStruct((_N, _D), jnp.float32),
    )(ego, partial, W1, b1.reshape(1, _D), W2, b2.reshape(1, _D))


def kernel(ego_embeddings, edge_index, edge_values, W1, b1, W2, b2):
    src = edge_index[0].astype(jnp.int32)
    dst = edge_index[1].astype(jnp.int32)
    ev = edge_values.astype(jnp.float32)
    zeros = jnp.zeros((_RPT, _D), jnp.float32)
    partial = _sc_segment_sum(ego_embeddings, src, dst, ev, zeros)
    return _tc_combine(ego_embeddings, partial, W1, b1, W2, b2)
